# baseline (speedup 1.0000x reference)
import functools

import jax
import jax.numpy as jnp
from jax import lax
from jax.experimental import pallas as pl
from jax.experimental.pallas import tpu as pltpu
from jax.experimental.pallas import tpu_sc as plsc

LANES = 16
IDX_W = 128
CHUNK = 256
NBUF = 2


def _sc_workers():
    try:
        info = plsc.get_sparse_core_info()
        return info.num_cores, info.num_subcores
    except Exception:
        return 2, 16


def kernel(ctr, table):
    batch, clicked = ctr.shape
    vocab, d = table.shape
    n_total = batch * clicked

    nc, ns = _sc_workers()
    nw = nc * ns
    per_w = n_total // nw
    assert per_w * nw == n_total and per_w % CHUNK == 0
    n_chunks = per_w // CHUNK
    assert n_chunks % NBUF == 0 and n_chunks >= 2 * NBUF
    n_outer = n_chunks // NBUF

    idx2d = ctr.reshape(n_total // CHUNK, CHUNK)

    mesh = plsc.VectorSubcoreMesh(core_axis_name="c", subcore_axis_name="s")

    @functools.partial(
        pl.kernel,
        out_type=jax.ShapeDtypeStruct((n_total, d), jnp.float32),
        mesh=mesh,
        scratch_types=[
            pltpu.VMEM((NBUF, 1, CHUNK), jnp.int32),
            pltpu.VMEM((NBUF, CHUNK, d), jnp.float32),
        ] + [pltpu.SemaphoreType.DMA] * (3 * NBUF),
    )
    def emb(idx_hbm, table_hbm, out_hbm, idx_v, rows_v, *sems):
        isem = sems[0:NBUF]
        gsem = sems[NBUF:2 * NBUF]
        osem = sems[2 * NBUF:3 * NBUF]
        wid = lax.axis_index("s") * nc + lax.axis_index("c")
        base = wid * per_w
        base_chunks = wid * n_chunks

        def idx_src(g):
            return idx_hbm.at[pl.ds(base_chunks + g, 1)]

        def out_dst(g):
            return out_hbm.at[pl.ds(base + g * CHUNK, CHUNK)]

        def start_idx(g, b):
            pltpu.async_copy(idx_src(g), idx_v.at[b], isem[b])

        def do_chunk(g, b, wait_out_g, prefetch_g):
            pltpu.make_async_copy(idx_src(g), idx_v.at[b], isem[b]).wait()
            for i in range(CHUNK // LANES):
                sl = (b, 0, pl.ds(i * LANES, LANES))
                v = idx_v[sl]
                idx_v[sl] = jnp.minimum(jnp.maximum(v, 0), vocab - 1)
            if wait_out_g is not None:
                pltpu.make_async_copy(rows_v.at[b], out_dst(wait_out_g),
                                      osem[b]).wait()
            pltpu.async_copy(table_hbm.at[idx_v.at[b, 0]],
                             rows_v.at[b], gsem[b]).wait()
            pltpu.async_copy(rows_v.at[b], out_dst(g), osem[b])
            if prefetch_g is not None:
                start_idx(prefetch_g, b)

        for b in range(NBUF):
            start_idx(b, b)
        for b in range(NBUF):
            do_chunk(b, b, None, NBUF + b)

        def body(outer, carry):
            g0 = outer * NBUF
            for b in range(NBUF):
                do_chunk(g0 + b, b, g0 + b - NBUF, g0 + b + NBUF)
            return carry

        lax.fori_loop(1, n_outer - 1, body, 0)

        g0 = (n_outer - 1) * NBUF
        for b in range(NBUF):
            do_chunk(g0 + b, b, g0 + b - NBUF, None)
        for b in range(NBUF):
            pltpu.make_async_copy(rows_v.at[b], out_dst(g0 + b), osem[b]).wait()

    out = emb(idx2d, table)
    return out.reshape(batch, clicked, d)


# speedup vs baseline: 1.0034x; 1.0034x over previous
import functools

import jax
import jax.numpy as jnp
from jax import lax
from jax.experimental import pallas as pl
from jax.experimental.pallas import tpu as pltpu
from jax.experimental.pallas import tpu_sc as plsc

LANES = 16
CHUNK = 256
NBUF = 2


def _sc_workers():
    try:
        info = plsc.get_sparse_core_info()
        return info.num_cores, info.num_subcores
    except Exception:
        return 2, 16


def kernel(ctr, table):
    batch, clicked = ctr.shape
    vocab, d = table.shape
    n_total = batch * clicked

    nc, ns = _sc_workers()
    nw = nc * ns
    per_w = n_total // nw
    assert per_w * nw == n_total and per_w % CHUNK == 0
    n_chunks = per_w // CHUNK
    assert n_chunks % NBUF == 0 and n_chunks >= 3 * NBUF

    idx2d = ctr.reshape(n_total // CHUNK, CHUNK)

    mesh = plsc.VectorSubcoreMesh(core_axis_name="c", subcore_axis_name="s")

    @functools.partial(
        pl.kernel,
        out_type=jax.ShapeDtypeStruct((n_total, d), jnp.float32),
        mesh=mesh,
        scratch_types=[
            pltpu.VMEM((NBUF, 1, CHUNK), jnp.int32),
            pltpu.VMEM((NBUF, CHUNK, d), jnp.float32),
        ] + [pltpu.SemaphoreType.DMA] * (3 * NBUF),
    )
    def emb(idx_hbm, table_hbm, out_hbm, idx_v, rows_v, *sems):
        isem = sems[0:NBUF]
        gsem = sems[NBUF:2 * NBUF]
        osem = sems[2 * NBUF:3 * NBUF]
        wid = lax.axis_index("s") * nc + lax.axis_index("c")
        base = wid * per_w
        base_chunks = wid * n_chunks

        def idx_src(g):
            return idx_hbm.at[pl.ds(base_chunks + g, 1)]

        def out_dst(g):
            return out_hbm.at[pl.ds(base + g * CHUNK, CHUNK)]

        def stage(h, b, first_use, do_write, do_prefetch):
            pltpu.make_async_copy(idx_src(h), idx_v.at[b], isem[b]).wait()
            for i in range(CHUNK // LANES):
                sl = (b, 0, pl.ds(i * LANES, LANES))
                v = idx_v[sl]
                idx_v[sl] = jnp.minimum(jnp.maximum(v, 0), vocab - 1)
            if not first_use:
                pltpu.make_async_copy(rows_v.at[b], out_dst(h - NBUF),
                                      osem[b]).wait()
            pltpu.async_copy(table_hbm.at[idx_v.at[b, 0]], rows_v.at[b],
                             gsem[b])
            o = 1 - b
            if do_write:
                pltpu.make_async_copy(table_hbm.at[pl.ds(0, CHUNK)],
                                      rows_v.at[o], gsem[o]).wait()
                pltpu.async_copy(rows_v.at[o], out_dst(h - 1), osem[o])
            if do_prefetch:
                pltpu.async_copy(idx_src(h + 1), idx_v.at[o], isem[o])

        for b in range(NBUF):
            pltpu.async_copy(idx_src(b), idx_v.at[b], isem[b])

        stage(0, 0, True, False, False)
        stage(1, 1, True, True, True)

        def body(k, carry):
            h0 = 2 * k
            stage(h0, 0, False, True, True)
            stage(h0 + 1, 1, False, True, True)
            return carry

        lax.fori_loop(1, n_chunks // 2 - 1, body, 0)

        stage(n_chunks - 2, 0, False, True, True)
        stage(n_chunks - 1, 1, False, True, False)

        pltpu.make_async_copy(table_hbm.at[pl.ds(0, CHUNK)], rows_v.at[1],
                              gsem[1]).wait()
        pltpu.async_copy(rows_v.at[1], out_dst(n_chunks - 1), osem[1])
        pltpu.make_async_copy(rows_v.at[0], out_dst(n_chunks - 2),
                              osem[0]).wait()
        pltpu.make_async_copy(rows_v.at[1], out_dst(n_chunks - 1),
                              osem[1]).wait()

    out = emb(idx2d, table)
    return out.reshape(batch, clicked, d)
